# baseline (speedup 1.0000x reference)
import functools

import jax
import jax.numpy as jnp
from jax import lax
from jax.experimental import pallas as pl
from jax.experimental.pallas import tpu as pltpu
from jax.experimental.pallas import tpu_sc as plsc

NUM_CORES = 2
NUM_SUBCORES = 16
NUM_WORKERS = NUM_CORES * NUM_SUBCORES

NUM_EMB = 1000000
NPAIR = NUM_EMB // 2
B = 4096 * 26
D = 64
L = 16

RC = 400
NRC = NUM_EMB // RC

CHUNK = 416
B_PER_W = B // NUM_WORKERS
NCHUNK = B_PER_W // CHUNK

SEL_I = 256


@jax.jit
def _sc_repack(weight):
    mesh = plsc.VectorSubcoreMesh(core_axis_name="c", subcore_axis_name="s")

    @functools.partial(
        pl.kernel,
        mesh=mesh,
        out_type=jax.ShapeDtypeStruct((NPAIR, 2 * D), jnp.float32),
        scratch_types=[
            pltpu.VMEM((RC, D), jnp.float32),
            pltpu.VMEM((RC // 2, 2 * D), jnp.float32),
            pltpu.SemaphoreType.DMA,
        ],
    )
    def k(table_hbm, out_hbm, stage_v, pack_v, sem):
        wid = lax.axis_index("s") * NUM_CORES + lax.axis_index("c")
        nk = (NRC - wid + NUM_WORKERS - 1) // NUM_WORKERS

        @pl.loop(0, nk)
        def _(kk):
            ci = wid + kk * NUM_WORKERS
            r0 = pl.multiple_of(ci * RC, 8)
            pltpu.async_copy(
                table_hbm.at[pl.ds(r0, RC)], stage_v, sem
            ).wait()

            @pl.loop(0, RC // 2, step=2)
            def _(q):
                for qq in range(2):
                    for h in range(2):
                        for c in range(D // L):
                            val = stage_v[2 * (q + qq) + h, pl.ds(c * L, L)]
                            pack_v[q + qq, pl.ds(h * D + c * L, L)] = val

            pltpu.async_copy(
                pack_v, out_hbm.at[pl.ds(pl.multiple_of(r0 // 2, 8), RC // 2)], sem
            ).wait()

    return k(weight)


@jax.jit
def _sc_gather_pairs(w2, idx2):
    mesh = plsc.VectorSubcoreMesh(core_axis_name="c", subcore_axis_name="s")

    @functools.partial(
        pl.kernel,
        mesh=mesh,
        out_type=jax.ShapeDtypeStruct((B, 2 * D), jnp.float32),
        scratch_types=[
            pltpu.VMEM((CHUNK,), jnp.int32),
            pltpu.VMEM((CHUNK, 2 * D), jnp.float32),
            pltpu.SemaphoreType.DMA,
        ],
    )
    def k(table_hbm, idx_hbm, out_hbm, idx_v, rows_v, sem):
        wid = lax.axis_index("s") * NUM_CORES + lax.axis_index("c")
        base = wid * B_PER_W
        for c in range(NCHUNK):
            off = base + c * CHUNK
            pltpu.sync_copy(idx_hbm.at[pl.ds(off, CHUNK)], idx_v)
            pltpu.async_copy(table_hbm.at[idx_v], rows_v, sem).wait()
            pltpu.sync_copy(rows_v, out_hbm.at[pl.ds(off, CHUNK)])

    return k(w2, idx2)


def _select_body(pairs_ref, par_ref, out_ref):
    pairs = pairs_ref[...]
    par = par_ref[...]
    out_ref[...] = jnp.where(par[:, :, None] == 0, pairs[:, :, :D], pairs[:, :, D:])


@functools.partial(jax.jit, static_argnums=(2, 3))
def _tc_select(pairs3, parity, nrows, ncols):
    return pl.pallas_call(
        _select_body,
        out_shape=jax.ShapeDtypeStruct((nrows, ncols, D), jnp.float32),
        grid=(nrows // SEL_I,),
        in_specs=[
            pl.BlockSpec((SEL_I, ncols, 2 * D), lambda i: (i, 0, 0)),
            pl.BlockSpec((SEL_I, ncols), lambda i: (i, 0)),
        ],
        out_specs=pl.BlockSpec((SEL_I, ncols, D), lambda i: (i, 0, 0)),
    )(pairs3, parity)


def kernel(x, weight):
    s = x.shape
    idx_flat = x.reshape(-1).astype(jnp.int32)
    w2 = _sc_repack(weight)
    pairs = _sc_gather_pairs(w2, idx_flat >> 1)
    parity = (x & 1).astype(jnp.int32)
    out = _tc_select(pairs.reshape(s[0], s[1], 2 * D), parity, s[0], s[1])
    return out


# speedup vs baseline: 1.5896x; 1.5896x over previous
import functools

import jax
import jax.numpy as jnp
from jax import lax
from jax.experimental import pallas as pl
from jax.experimental.pallas import tpu as pltpu
from jax.experimental.pallas import tpu_sc as plsc

NUM_CORES = 2
NUM_SUBCORES = 16
NUM_WORKERS = NUM_CORES * NUM_SUBCORES

NUM_EMB = 1000000
NPAIR = NUM_EMB // 2
B = 4096 * 26
D = 64
L = 16

RC = 400
NRC = NUM_EMB // RC

CHUNK = 416
B_PER_W = B // NUM_WORKERS
NCHUNK = B_PER_W // CHUNK

SEL_I = 256


@jax.jit
def _sc_repack(weight):
    mesh = plsc.VectorSubcoreMesh(core_axis_name="c", subcore_axis_name="s")

    @functools.partial(
        pl.kernel,
        mesh=mesh,
        out_type=jax.ShapeDtypeStruct((NPAIR, 2 * D), jnp.float32),
        scratch_types=[
            pltpu.VMEM((RC, D), jnp.float32),
            pltpu.VMEM((RC // 2, 2 * D), jnp.float32),
            pltpu.SemaphoreType.DMA,
        ],
    )
    def k(table_hbm, out_hbm, stage_v, pack_v, sem):
        wid = lax.axis_index("s") * NUM_CORES + lax.axis_index("c")
        nk = (NRC - wid + NUM_WORKERS - 1) // NUM_WORKERS

        @pl.loop(0, nk)
        def _(kk):
            ci = wid + kk * NUM_WORKERS
            r0 = pl.multiple_of(ci * RC, 8)
            pltpu.async_copy(
                table_hbm.at[pl.ds(r0, RC)], stage_v, sem
            ).wait()

            @pl.loop(0, RC // 2, step=2)
            def _(q):
                for qq in range(2):
                    for h in range(2):
                        for c in range(D // L):
                            val = stage_v[2 * (q + qq) + h, pl.ds(c * L, L)]
                            pack_v[q + qq, pl.ds(h * D + c * L, L)] = val

            pltpu.async_copy(
                pack_v, out_hbm.at[pl.ds(pl.multiple_of(r0 // 2, 8), RC // 2)], sem
            ).wait()

    return k(weight)


@jax.jit
def _sc_gather_pairs(w2, idx2):
    mesh = plsc.VectorSubcoreMesh(core_axis_name="c", subcore_axis_name="s")

    @functools.partial(
        pl.kernel,
        mesh=mesh,
        out_type=jax.ShapeDtypeStruct((B, 2 * D), jnp.float32),
        scratch_types=[
            pltpu.VMEM((CHUNK,), jnp.int32),
            pltpu.VMEM((CHUNK, 2 * D), jnp.float32),
            pltpu.SemaphoreType.DMA,
        ],
    )
    def k(table_hbm, idx_hbm, out_hbm, idx_v, rows_v, sem):
        wid = lax.axis_index("s") * NUM_CORES + lax.axis_index("c")
        base = wid * B_PER_W
        for c in range(NCHUNK):
            off = base + c * CHUNK
            pltpu.sync_copy(idx_hbm.at[pl.ds(off, CHUNK)], idx_v)
            pltpu.async_copy(table_hbm.at[idx_v], rows_v, sem).wait()
            pltpu.sync_copy(rows_v, out_hbm.at[pl.ds(off, CHUNK)])

    return k(w2, idx2)


def _select_body(pairs_ref, par_ref, out_ref):
    pairs = pairs_ref[...]
    par = par_ref[...]
    out_ref[...] = jnp.where(par[:, :, None] == 0, pairs[:, :, :D], pairs[:, :, D:])


@functools.partial(jax.jit, static_argnums=(2, 3))
def _tc_select(pairs3, parity, nrows, ncols):
    return pl.pallas_call(
        _select_body,
        out_shape=jax.ShapeDtypeStruct((nrows, ncols, D), jnp.float32),
        grid=(nrows // SEL_I,),
        in_specs=[
            pl.BlockSpec((SEL_I, ncols, 2 * D), lambda i: (i, 0, 0)),
            pl.BlockSpec((SEL_I, ncols), lambda i: (i, 0)),
        ],
        out_specs=pl.BlockSpec((SEL_I, ncols, D), lambda i: (i, 0, 0)),
    )(pairs3, parity)


def kernel(x, weight):
    s = x.shape
    idx_flat = x.reshape(-1).astype(jnp.int32)
    w2 = weight.reshape(NPAIR, 2 * D)
    pairs = _sc_gather_pairs(w2, idx_flat >> 1)
    parity = (x & 1).astype(jnp.int32)
    out = _tc_select(pairs.reshape(s[0], s[1], 2 * D), parity, s[0], s[1])
    return out


# speedup vs baseline: 1.7157x; 1.0793x over previous
import functools

import jax
import jax.numpy as jnp
from jax import lax
from jax.experimental import pallas as pl
from jax.experimental.pallas import tpu as pltpu
from jax.experimental.pallas import tpu_sc as plsc

NUM_CORES = 2
NUM_SUBCORES = 16
NUM_WORKERS = NUM_CORES * NUM_SUBCORES

NUM_EMB = 1000000
NPAIR = NUM_EMB // 2
B = 4096 * 26
D = 64
L = 16

RC = 400
NRC = NUM_EMB // RC

CHUNK = 416
B_PER_W = B // NUM_WORKERS
NCHUNK = B_PER_W // CHUNK

SEL_I = 256


@jax.jit
def _sc_repack(weight):
    mesh = plsc.VectorSubcoreMesh(core_axis_name="c", subcore_axis_name="s")

    @functools.partial(
        pl.kernel,
        mesh=mesh,
        out_type=jax.ShapeDtypeStruct((NPAIR, 2 * D), jnp.float32),
        scratch_types=[
            pltpu.VMEM((RC, D), jnp.float32),
            pltpu.VMEM((RC // 2, 2 * D), jnp.float32),
            pltpu.SemaphoreType.DMA,
        ],
    )
    def k(table_hbm, out_hbm, stage_v, pack_v, sem):
        wid = lax.axis_index("s") * NUM_CORES + lax.axis_index("c")
        nk = (NRC - wid + NUM_WORKERS - 1) // NUM_WORKERS

        @pl.loop(0, nk)
        def _(kk):
            ci = wid + kk * NUM_WORKERS
            r0 = pl.multiple_of(ci * RC, 8)
            pltpu.async_copy(
                table_hbm.at[pl.ds(r0, RC)], stage_v, sem
            ).wait()

            @pl.loop(0, RC // 2, step=2)
            def _(q):
                for qq in range(2):
                    for h in range(2):
                        for c in range(D // L):
                            val = stage_v[2 * (q + qq) + h, pl.ds(c * L, L)]
                            pack_v[q + qq, pl.ds(h * D + c * L, L)] = val

            pltpu.async_copy(
                pack_v, out_hbm.at[pl.ds(pl.multiple_of(r0 // 2, 8), RC // 2)], sem
            ).wait()

    return k(weight)


@jax.jit
def _sc_gather_pairs(w2, idx2):
    mesh = plsc.VectorSubcoreMesh(core_axis_name="c", subcore_axis_name="s")

    @functools.partial(
        pl.kernel,
        mesh=mesh,
        out_type=jax.ShapeDtypeStruct((4096, 26, 2 * D), jnp.float32),
        scratch_types=[
            pltpu.VMEM((CHUNK,), jnp.int32),
            pltpu.VMEM((CHUNK, 2 * D), jnp.float32),
            pltpu.SemaphoreType.DMA,
        ],
    )
    def k(table_hbm, idx_hbm, out_hbm, idx_v, rows_v, sem):
        wid = lax.axis_index("s") * NUM_CORES + lax.axis_index("c")
        base = wid * B_PER_W
        ci = CHUNK // 26
        for c in range(NCHUNK):
            off = base + c * CHUNK
            pltpu.sync_copy(idx_hbm.at[pl.ds(off, CHUNK)], idx_v)
            pltpu.async_copy(table_hbm.at[idx_v], rows_v, sem).wait()
            pltpu.sync_copy(
                rows_v.reshape(ci, 26, 2 * D),
                out_hbm.at[pl.ds(off // 26, ci)],
            )

    return k(w2, idx2)


def _select_body(pairs_ref, par_ref, out_ref):
    pairs = pairs_ref[...]
    par = par_ref[...]
    out_ref[...] = jnp.where(par[:, :, None] == 0, pairs[:, :, :D], pairs[:, :, D:])


@functools.partial(jax.jit, static_argnums=(2, 3))
def _tc_select(pairs3, parity, nrows, ncols):
    return pl.pallas_call(
        _select_body,
        out_shape=jax.ShapeDtypeStruct((nrows, ncols, D), jnp.float32),
        grid=(nrows // SEL_I,),
        in_specs=[
            pl.BlockSpec((SEL_I, ncols, 2 * D), lambda i: (i, 0, 0)),
            pl.BlockSpec((SEL_I, ncols), lambda i: (i, 0)),
        ],
        out_specs=pl.BlockSpec((SEL_I, ncols, D), lambda i: (i, 0, 0)),
    )(pairs3, parity)


def kernel(x, weight):
    s = x.shape
    idx_flat = x.reshape(-1).astype(jnp.int32)
    w2 = (weight * jnp.float32(1.0)).reshape(NPAIR, 2 * D)
    pairs3 = _sc_gather_pairs(w2, idx_flat >> 1)
    parity = (x & 1).astype(jnp.int32)
    out = _tc_select(pairs3, parity, s[0], s[1])
    return out
